# baseline (speedup 1.0000x reference)
import functools

import jax
import jax.numpy as jnp
import numpy as np
from jax import lax
from jax.experimental import pallas as pl
from jax.experimental.pallas import tpu as pltpu
from jax.experimental.pallas import tpu_sc as plsc

_TW = 10
_TRES = 100
_NX = 64
_NY = 64
_B = 16
_N = _NX * _NY
_E = 94500
_CHUNK0 = 3576
_CHUNKM = 6024
_CHUNKL = 6588
_BUF = 6608


def _row_pattern(trep):
    di_lo, di_hi = max(-2, -trep), min(2, 63 - trep)
    s, d = [], []
    for j in range(64):
        for di in range(di_lo, di_hi + 1):
            for dj in range(max(-2, -j), min(2, 63 - j) + 1):
                if di == 0 and dj == 0:
                    continue
                s.append(j)
                d.append(di * 64 + j + dj)
    return np.asarray(s, np.int32), np.asarray(d, np.int32)


def _chunked(a):
    n = len(a)
    nch = -(-n // 16)
    return (np.concatenate([a[min(16 * u, n - 16):min(16 * u, n - 16) + 16]
                            for u in range(nch)]), n, nch)


def _build_tables():
    parts, meta, off = [], {}, 0
    for trep in (0, 1, 2, 62, 63):
        s, d = _row_pattern(trep)
        cs, n, nch = _chunked(s)
        cd, _, _ = _chunked(d)
        meta[trep] = (off, off + nch * 16, n, nch)
        parts.extend([cs, cd])
        off += 2 * nch * 16
    return np.concatenate(parts), meta


_TBL_NP, _TBL_META = _build_tables()
_TBL_LEN = len(_TBL_NP)


@functools.lru_cache(maxsize=None)
def _edge_sc_fn():
    mesh = plsc.VectorSubcoreMesh(core_axis_name="c", subcore_axis_name="s")

    @functools.partial(
        pl.kernel,
        out_type=jax.ShapeDtypeStruct((2 * _B * _E,), jnp.int32),
        mesh=mesh,
        scratch_types=[pltpu.VMEM((_TBL_LEN,), jnp.int32),
                       pltpu.VMEM((_BUF,), jnp.int32),
                       pltpu.VMEM((_BUF,), jnp.int32),
                       pltpu.VMEM((_BUF,), jnp.int32),
                       pltpu.VMEM((_BUF,), jnp.int32)],
    )
    def _edge_sc(tbl_hbm, edges_out, tbl, src_a, dst_a, src_b, dst_b):
        tid = lax.axis_index("s") * 2 + lax.axis_index("c")

        @pl.when(tid < 16)
        def _():
            k = tid
            pltpu.sync_copy(tbl_hbm, tbl)
            i_start = jnp.where(k == 0, 0, 4 * k - 1)
            i_end = jnp.where(k == 0, 3, jnp.where(k == 15, 64, 4 * k + 3))
            tile_base = (314 * (5 * i_start - jnp.minimum(i_start, 2)
                               - jnp.minimum(i_start, 1)) - 64 * i_start)

            def emit_row(rb, base_i, soff, doff, n, nch):
                def chunk(u, carry):
                    sv = tbl[pl.ds(soff + u * 16, 16)] + base_i
                    src_a[pl.ds(rb + u * 16, 16)] = sv
                    src_b[pl.ds(rb + 4 + u * 16, 16)] = sv
                    dv = tbl[pl.ds(doff + u * 16, 16)] + base_i
                    dst_a[pl.ds(rb + u * 16, 16)] = dv
                    dst_b[pl.ds(rb + 4 + u * 16, 16)] = dv
                    return carry

                lax.fori_loop(0, nch - 1, chunk, jnp.int32(0))
                last = nch - 1
                sv = tbl[pl.ds(soff + last * 16, 16)] + base_i
                src_a[pl.ds(rb + n - 16, 16)] = sv
                src_b[pl.ds(rb + 4 + n - 16, 16)] = sv
                dv = tbl[pl.ds(doff + last * 16, 16)] + base_i
                dst_a[pl.ds(rb + n - 16, 16)] = dv
                dst_b[pl.ds(rb + 4 + n - 16, 16)] = dv

            def vbody(v, carry):
                i = i_start + v

                @pl.when(i < i_end)
                def _():
                    wr = (5 * i - jnp.minimum(i, 2) - jnp.minimum(i, 1)
                          - jnp.maximum(i - 62, 0))
                    rb = 314 * wr - 64 * i - tile_base
                    base_i = i * 64
                    for trep, (soff, doff, n, nch) in _TBL_META.items():
                        if trep == 2:
                            cond = (i >= 2) & (i <= 61)
                        else:
                            cond = i == trep
                        pl.when(cond)(
                            lambda so=soff, do=doff, nn=n, nc=nch:
                            emit_row(rb, base_i, so, do, nn, nc))
                return carry

            lax.fori_loop(0, 5, vbody, jnp.int32(0))

            a_k = jnp.where(k == 0, 0, 3576 + 6024 * (k - 1))
            clen = jnp.where(k == 0, _CHUNK0,
                             jnp.where(k == 15, _CHUNKL, _CHUNKM))

            s2off, d2off = _TBL_META[2][0], _TBL_META[2][1]
            s0off, d0off = _TBL_META[0][0], _TBL_META[0][1]
            is_last = k == 15
            soff_nx = jnp.where(is_last, s0off, s2off)
            doff_nx = jnp.where(is_last, d0off, d2off)
            add_nx = jnp.where(is_last, _N, 64 * (4 * k + 3))
            nx_s = tbl[pl.ds(soff_nx, 16)] + add_nx
            nx_d = tbl[pl.ds(doff_nx, 16)] + add_nx
            src_b[pl.ds(clen + 4, 16)] = nx_s
            dst_b[pl.ds(clen + 4, 16)] = nx_d

            @pl.when(is_last)
            def _():
                src_a[pl.ds(_CHUNKL, 16)] = nx_s
                dst_a[pl.ds(_CHUNKL, 16)] = nx_d

            def add_all(buf, amt):
                def body(u, c2):
                    buf[pl.ds(u * 16, 16)] = buf[pl.ds(u * 16, 16)] + amt
                    return c2
                lax.fori_loop(0, _BUF // 16, body, jnp.int32(0))

            def pbody(p, carry):
                @pl.when(p > 0)
                def _():
                    add_all(src_a, jnp.int32(2 * _N))
                    add_all(dst_a, jnp.int32(2 * _N))

                ebase = p * (2 * _E) + a_k

                @pl.when(k == 0)
                def _():
                    pltpu.sync_copy(src_a.at[pl.ds(0, _CHUNK0)],
                                    edges_out.at[pl.ds(ebase, _CHUNK0)])
                    pltpu.sync_copy(dst_a.at[pl.ds(0, _CHUNK0)],
                                    edges_out.at[pl.ds(_B * _E + ebase,
                                                       _CHUNK0)])

                @pl.when((k >= 1) & (k < 15))
                def _():
                    pltpu.sync_copy(src_a.at[pl.ds(0, _CHUNKM)],
                                    edges_out.at[pl.ds(ebase, _CHUNKM)])
                    pltpu.sync_copy(dst_a.at[pl.ds(0, _CHUNKM)],
                                    edges_out.at[pl.ds(_B * _E + ebase,
                                                       _CHUNKM)])

                @pl.when(k == 15)
                def _():
                    pltpu.sync_copy(src_a.at[pl.ds(0, _CHUNKL + 4)],
                                    edges_out.at[pl.ds(ebase, _CHUNKL + 4)])
                    pltpu.sync_copy(dst_a.at[pl.ds(0, _CHUNKL + 4)],
                                    edges_out.at[pl.ds(_B * _E + ebase,
                                                       _CHUNKL + 4)])

                amt = jnp.where(p == 0, jnp.int32(_N), jnp.int32(2 * _N))
                add_all(src_b, amt)
                add_all(dst_b, amt)

                obase = p * (2 * _E) + (_E + 4) + a_k

                @pl.when(k == 0)
                def _():
                    pltpu.sync_copy(src_b.at[pl.ds(8, _CHUNK0)],
                                    edges_out.at[pl.ds(obase, _CHUNK0)])
                    pltpu.sync_copy(dst_b.at[pl.ds(8, _CHUNK0)],
                                    edges_out.at[pl.ds(_B * _E + obase,
                                                       _CHUNK0)])

                @pl.when((k >= 1) & (k < 15))
                def _():
                    pltpu.sync_copy(src_b.at[pl.ds(8, _CHUNKM)],
                                    edges_out.at[pl.ds(obase, _CHUNKM)])
                    pltpu.sync_copy(dst_b.at[pl.ds(8, _CHUNKM)],
                                    edges_out.at[pl.ds(_B * _E + obase,
                                                       _CHUNKM)])

                @pl.when(is_last & (p < 7))
                def _():
                    pltpu.sync_copy(src_b.at[pl.ds(8, _CHUNKL)],
                                    edges_out.at[pl.ds(obase, _CHUNKL)])
                    pltpu.sync_copy(dst_b.at[pl.ds(8, _CHUNKL)],
                                    edges_out.at[pl.ds(_B * _E + obase,
                                                       _CHUNKL)])

                @pl.when(is_last & (p == 7))
                def _():
                    pltpu.sync_copy(src_b.at[pl.ds(8, _CHUNKL - 4)],
                                    edges_out.at[pl.ds(obase, _CHUNKL - 4)])
                    pltpu.sync_copy(dst_b.at[pl.ds(8, _CHUNKL - 4)],
                                    edges_out.at[pl.ds(_B * _E + obase,
                                                       _CHUNKL - 4)])
                return carry

            lax.fori_loop(0, _B // 2, pbody, jnp.int32(0))

    return _edge_sc


def _fused_body(steps_ref, t_ref, grid_ref, d_ref, l_ref,
                u_ref, y_ref, pos_ref, batch_ref):
    b = pl.program_id(0)
    eye = jnp.eye(_TW, dtype=jnp.float32)
    dims = (((0,), (0,)), ((), ()))
    u_ref[...] = lax.dot_general(d_ref[0], eye, dims,
                                 precision=lax.Precision.HIGHEST)
    y_ref[...] = lax.dot_general(l_ref[0], eye, dims,
                                 precision=lax.Precision.HIGHEST)
    s = steps_ref[b, 0]
    tv = t_ref[s, 0]
    pos_ref[...] = jnp.concatenate(
        [jnp.full((_N, 1), tv, jnp.float32), grid_ref[...]], axis=1)
    batch_ref[...] = jnp.full((_N,), b, jnp.int32)


def _fused(steps2, t2, grid_nt, d3, l3):
    return pl.pallas_call(
        _fused_body,
        grid=(_B,),
        in_specs=[pl.BlockSpec(memory_space=pltpu.SMEM),
                  pl.BlockSpec(memory_space=pltpu.SMEM),
                  pl.BlockSpec((_N, 2), lambda b: (0, 0)),
                  pl.BlockSpec((1, _TW, _N), lambda b: (b, 0, 0)),
                  pl.BlockSpec((1, _TW, _N), lambda b: (b, 0, 0))],
        out_specs=[pl.BlockSpec((_N, _TW), lambda b: (b, 0)),
                   pl.BlockSpec((_N, _TW), lambda b: (b, 0)),
                   pl.BlockSpec((_N, 3), lambda b: (b, 0)),
                   pl.BlockSpec((_N,), lambda b: (b,))],
        out_shape=[jax.ShapeDtypeStruct((_B * _N, _TW), jnp.float32),
                   jax.ShapeDtypeStruct((_B * _N, _TW), jnp.float32),
                   jax.ShapeDtypeStruct((_B * _N, 3), jnp.float32),
                   jax.ShapeDtypeStruct((_B * _N,), jnp.int32)],
    )(steps2, t2, grid_nt, d3, l3)


def kernel(data, labels, steps):
    b, tw, nx, ny = data.shape
    d3 = data.reshape(b, tw, _N)
    l3 = labels.reshape(b, tw, _N)

    edges = _edge_sc_fn()(jnp.asarray(_TBL_NP)).reshape(2, _B * _E)

    t_tab = jnp.linspace(0.0, 1.0, _TRES).astype(jnp.float32).reshape(_TRES, 1)
    xs = jnp.linspace(0.0, 1.0, _NX)
    ys = jnp.linspace(0.0, 1.0, _NY)
    gx, gy = jnp.meshgrid(xs, ys, indexing="ij")
    grid_nt = jnp.stack((gx.reshape(_N), gy.reshape(_N)), 1).astype(jnp.float32)

    u, y, pos, batch = _fused(steps.reshape(_B, 1), t_tab, grid_nt, d3, l3)

    return (u, edges, y, pos, batch)


# speedup vs baseline: 1.0009x; 1.0009x over previous
import functools

import jax
import jax.numpy as jnp
import numpy as np
from jax import lax
from jax.experimental import pallas as pl
from jax.experimental.pallas import tpu as pltpu
from jax.experimental.pallas import tpu_sc as plsc

_TW = 10
_TRES = 100
_NX = 64
_NY = 64
_B = 16
_N = _NX * _NY
_E = 94500
_CHUNK0 = 3576
_CHUNKM = 6024
_CHUNKL = 6588
_BUF = 6608


def _row_pattern(trep):
    di_lo, di_hi = max(-2, -trep), min(2, 63 - trep)
    s, d = [], []
    for j in range(64):
        for di in range(di_lo, di_hi + 1):
            for dj in range(max(-2, -j), min(2, 63 - j) + 1):
                if di == 0 and dj == 0:
                    continue
                s.append(j)
                d.append(di * 64 + j + dj)
    return np.asarray(s, np.int32), np.asarray(d, np.int32)


def _chunked(a):
    n = len(a)
    nch = -(-n // 16)
    return (np.concatenate([a[min(16 * u, n - 16):min(16 * u, n - 16) + 16]
                            for u in range(nch)]), n, nch)


def _build_tables():
    parts, meta, off = [], {}, 0
    for trep in (0, 1, 2, 62, 63):
        s, d = _row_pattern(trep)
        cs, n, nch = _chunked(s)
        cd, _, _ = _chunked(d)
        meta[trep] = (off, off + nch * 16, n, nch)
        parts.extend([cs, cd])
        off += 2 * nch * 16
    return np.concatenate(parts), meta


_TBL_NP, _TBL_META = _build_tables()
_TBL_LEN = len(_TBL_NP)


@functools.lru_cache(maxsize=None)
def _edge_sc_fn():
    mesh = plsc.VectorSubcoreMesh(core_axis_name="c", subcore_axis_name="s")

    @functools.partial(
        pl.kernel,
        out_type=jax.ShapeDtypeStruct((2 * _B * _E,), jnp.int32),
        mesh=mesh,
        scratch_types=[pltpu.VMEM((_TBL_LEN,), jnp.int32),
                       pltpu.VMEM((_BUF,), jnp.int32),
                       pltpu.VMEM((_BUF,), jnp.int32),
                       pltpu.VMEM((_BUF,), jnp.int32),
                       pltpu.VMEM((_BUF,), jnp.int32),
                       pltpu.SemaphoreType.DMA],
    )
    def _edge_sc(tbl_hbm, edges_out, tbl, src_a, dst_a, src_b, dst_b, sem):
        tid = lax.axis_index("s") * 2 + lax.axis_index("c")

        @pl.when(tid < 16)
        def _():
            k = tid
            pltpu.sync_copy(tbl_hbm, tbl)
            i_start = jnp.where(k == 0, 0, 4 * k - 1)
            i_end = jnp.where(k == 0, 3, jnp.where(k == 15, 64, 4 * k + 3))
            tile_base = (314 * (5 * i_start - jnp.minimum(i_start, 2)
                               - jnp.minimum(i_start, 1)) - 64 * i_start)

            def emit_row(rb, base_i, soff, doff, n, nch):
                def chunk(u, carry):
                    sv = tbl[pl.ds(soff + u * 16, 16)] + base_i
                    src_a[pl.ds(rb + u * 16, 16)] = sv
                    src_b[pl.ds(rb + 4 + u * 16, 16)] = sv
                    dv = tbl[pl.ds(doff + u * 16, 16)] + base_i
                    dst_a[pl.ds(rb + u * 16, 16)] = dv
                    dst_b[pl.ds(rb + 4 + u * 16, 16)] = dv
                    return carry

                lax.fori_loop(0, nch - 1, chunk, jnp.int32(0))
                last = nch - 1
                sv = tbl[pl.ds(soff + last * 16, 16)] + base_i
                src_a[pl.ds(rb + n - 16, 16)] = sv
                src_b[pl.ds(rb + 4 + n - 16, 16)] = sv
                dv = tbl[pl.ds(doff + last * 16, 16)] + base_i
                dst_a[pl.ds(rb + n - 16, 16)] = dv
                dst_b[pl.ds(rb + 4 + n - 16, 16)] = dv

            def vbody(v, carry):
                i = i_start + v

                @pl.when(i < i_end)
                def _():
                    wr = (5 * i - jnp.minimum(i, 2) - jnp.minimum(i, 1)
                          - jnp.maximum(i - 62, 0))
                    rb = 314 * wr - 64 * i - tile_base
                    base_i = i * 64
                    for trep, (soff, doff, n, nch) in _TBL_META.items():
                        if trep == 2:
                            cond = (i >= 2) & (i <= 61)
                        else:
                            cond = i == trep
                        pl.when(cond)(
                            lambda so=soff, do=doff, nn=n, nc=nch:
                            emit_row(rb, base_i, so, do, nn, nc))
                return carry

            lax.fori_loop(0, 5, vbody, jnp.int32(0))

            a_k = jnp.where(k == 0, 0, 3576 + 6024 * (k - 1))
            clen = jnp.where(k == 0, _CHUNK0,
                             jnp.where(k == 15, _CHUNKL, _CHUNKM))

            s2off, d2off = _TBL_META[2][0], _TBL_META[2][1]
            s0off, d0off = _TBL_META[0][0], _TBL_META[0][1]
            is_last = k == 15
            soff_nx = jnp.where(is_last, s0off, s2off)
            doff_nx = jnp.where(is_last, d0off, d2off)
            add_nx = jnp.where(is_last, _N, 64 * (4 * k + 3))
            nx_s = tbl[pl.ds(soff_nx, 16)] + add_nx
            nx_d = tbl[pl.ds(doff_nx, 16)] + add_nx
            src_b[pl.ds(clen + 4, 16)] = nx_s
            dst_b[pl.ds(clen + 4, 16)] = nx_d

            @pl.when(is_last)
            def _():
                src_a[pl.ds(_CHUNKL, 16)] = nx_s
                dst_a[pl.ds(_CHUNKL, 16)] = nx_d

            def pbody(p, carry):
                @pl.when(p == 0)
                def _():
                    def b0(u, c2):
                        src_b[pl.ds(u * 16, 16)] = (
                            src_b[pl.ds(u * 16, 16)] + _N)
                        dst_b[pl.ds(u * 16, 16)] = (
                            dst_b[pl.ds(u * 16, 16)] + _N)
                        return c2
                    lax.fori_loop(0, _BUF // 16, b0, jnp.int32(0))

                @pl.when(p > 0)
                def _():
                    def bp(u, c2):
                        src_a[pl.ds(u * 16, 16)] = (
                            src_a[pl.ds(u * 16, 16)] + 2 * _N)
                        dst_a[pl.ds(u * 16, 16)] = (
                            dst_a[pl.ds(u * 16, 16)] + 2 * _N)
                        src_b[pl.ds(u * 16, 16)] = (
                            src_b[pl.ds(u * 16, 16)] + 2 * _N)
                        dst_b[pl.ds(u * 16, 16)] = (
                            dst_b[pl.ds(u * 16, 16)] + 2 * _N)
                        return c2
                    lax.fori_loop(0, _BUF // 16, bp, jnp.int32(0))

                ebase = p * (2 * _E) + a_k
                obase = p * (2 * _E) + (_E + 4) + a_k

                def fire4(alen, olen):
                    c1 = pltpu.async_copy(
                        src_a.at[pl.ds(0, alen)],
                        edges_out.at[pl.ds(ebase, alen)], sem)
                    c2 = pltpu.async_copy(
                        dst_a.at[pl.ds(0, alen)],
                        edges_out.at[pl.ds(_B * _E + ebase, alen)], sem)
                    c3 = pltpu.async_copy(
                        src_b.at[pl.ds(8, olen)],
                        edges_out.at[pl.ds(obase, olen)], sem)
                    c4 = pltpu.async_copy(
                        dst_b.at[pl.ds(8, olen)],
                        edges_out.at[pl.ds(_B * _E + obase, olen)], sem)
                    c1.wait()
                    c2.wait()
                    c3.wait()
                    c4.wait()

                @pl.when(k == 0)
                def _():
                    fire4(_CHUNK0, _CHUNK0)

                @pl.when((k >= 1) & (k < 15))
                def _():
                    fire4(_CHUNKM, _CHUNKM)

                @pl.when(is_last & (p < 7))
                def _():
                    fire4(_CHUNKL + 4, _CHUNKL)

                @pl.when(is_last & (p == 7))
                def _():
                    fire4(_CHUNKL + 4, _CHUNKL - 4)
                return carry

            lax.fori_loop(0, _B // 2, pbody, jnp.int32(0))

    return _edge_sc


def _fused_body(steps_ref, t_ref, grid_ref, d_ref, l_ref,
                u_ref, y_ref, pos_ref, batch_ref):
    b = pl.program_id(0)
    eye = jnp.eye(_TW, dtype=jnp.float32)
    dims = (((0,), (0,)), ((), ()))
    u_ref[...] = lax.dot_general(d_ref[0], eye, dims,
                                 precision=lax.Precision.HIGHEST)
    y_ref[...] = lax.dot_general(l_ref[0], eye, dims,
                                 precision=lax.Precision.HIGHEST)
    s = steps_ref[b, 0]
    tv = t_ref[s, 0]
    pos_ref[...] = jnp.concatenate(
        [jnp.full((_N, 1), tv, jnp.float32), grid_ref[...]], axis=1)
    batch_ref[...] = jnp.full((_N,), b, jnp.int32)


def _fused(steps2, t2, grid_nt, d3, l3):
    return pl.pallas_call(
        _fused_body,
        grid=(_B,),
        in_specs=[pl.BlockSpec(memory_space=pltpu.SMEM),
                  pl.BlockSpec(memory_space=pltpu.SMEM),
                  pl.BlockSpec((_N, 2), lambda b: (0, 0)),
                  pl.BlockSpec((1, _TW, _N), lambda b: (b, 0, 0)),
                  pl.BlockSpec((1, _TW, _N), lambda b: (b, 0, 0))],
        out_specs=[pl.BlockSpec((_N, _TW), lambda b: (b, 0)),
                   pl.BlockSpec((_N, _TW), lambda b: (b, 0)),
                   pl.BlockSpec((_N, 3), lambda b: (b, 0)),
                   pl.BlockSpec((_N,), lambda b: (b,))],
        out_shape=[jax.ShapeDtypeStruct((_B * _N, _TW), jnp.float32),
                   jax.ShapeDtypeStruct((_B * _N, _TW), jnp.float32),
                   jax.ShapeDtypeStruct((_B * _N, 3), jnp.float32),
                   jax.ShapeDtypeStruct((_B * _N,), jnp.int32)],
    )(steps2, t2, grid_nt, d3, l3)


def kernel(data, labels, steps):
    b, tw, nx, ny = data.shape
    d3 = data.reshape(b, tw, _N)
    l3 = labels.reshape(b, tw, _N)

    edges = _edge_sc_fn()(jnp.asarray(_TBL_NP)).reshape(2, _B * _E)

    t_tab = jnp.linspace(0.0, 1.0, _TRES).astype(jnp.float32).reshape(_TRES, 1)
    xs = jnp.linspace(0.0, 1.0, _NX)
    ys = jnp.linspace(0.0, 1.0, _NY)
    gx, gy = jnp.meshgrid(xs, ys, indexing="ij")
    grid_nt = jnp.stack((gx.reshape(_N), gy.reshape(_N)), 1).astype(jnp.float32)

    u, y, pos, batch = _fused(steps.reshape(_B, 1), t_tab, grid_nt, d3, l3)

    return (u, edges, y, pos, batch)
